# baseline (speedup 1.0000x reference)
import functools

import jax
import jax.numpy as jnp
from jax import lax
from jax.experimental import pallas as pl
from jax.experimental.pallas import tpu as pltpu
from jax.experimental.pallas import tpu_sc as plsc

N_BANK = 1_000_000
D_DIM = 128
B_TOK = 16384

_NC = 2
_NS = 16
_NW = _NC * _NS
_B_PER_W = B_TOK // _NW
_BLK = _B_PER_W // 2

_mesh = plsc.VectorSubcoreMesh(core_axis_name="c", subcore_axis_name="s")


@functools.partial(
    pl.kernel,
    mesh=_mesh,
    out_type=jax.ShapeDtypeStruct((B_TOK, D_DIM), jnp.float32),
    scratch_types=[
        pltpu.VMEM((_B_PER_W,), jnp.int32),
        pltpu.VMEM((_BLK, D_DIM), jnp.float32),
        pltpu.VMEM((_BLK, D_DIM), jnp.float32),
        pltpu.SemaphoreType.DMA,
        pltpu.SemaphoreType.DMA,
    ],
)
def _gather_rows(idx_hbm, mbt_hbm, out_hbm, idx_v, rows0, rows1, gsem, osem):
    wid = lax.axis_index("s") * _NC + lax.axis_index("c")
    base = wid * _B_PER_W
    pltpu.sync_copy(idx_hbm.at[pl.ds(base, _B_PER_W)], idx_v)

    g0 = pltpu.async_copy(mbt_hbm.at[idx_v.at[pl.ds(0, _BLK)]], rows0, gsem)
    g1 = pltpu.async_copy(mbt_hbm.at[idx_v.at[pl.ds(_BLK, _BLK)]], rows1, gsem)
    g0.wait()
    s0 = pltpu.async_copy(rows0, out_hbm.at[pl.ds(base, _BLK)], osem)
    g1.wait()
    s1 = pltpu.async_copy(rows1, out_hbm.at[pl.ds(base + _BLK, _BLK)], osem)
    s0.wait()
    s1.wait()


def kernel(index, membank):
    mbt = membank.T
    out_t = _gather_rows(index, mbt)
    return out_t.T
